# Initial kernel scaffold; baseline (speedup 1.0000x reference)
import jax
import jax.numpy as jnp
from jax.experimental import pallas as pl


def kernel(indices, weight):
    raise NotImplementedError("write your pallas kernel here")


# speedup vs baseline: 1.4675x; 1.4675x over previous
import functools

import jax
import jax.numpy as jnp
from jax import lax
from jax.experimental import pallas as pl
from jax.experimental.pallas import tpu as pltpu
from jax.experimental.pallas import tpu_sc as plsc

N_SYMBOLS = 1000000
OUTPUT_DIM = 32
B_TOTAL = 16384 * 20

_NC = 2
_NS = 16
_NW = _NC * _NS

_B_PER_W = B_TOTAL // _NW
_CHUNK = 1024
_N_CHUNKS = _B_PER_W // _CHUNK


def _embed_body(idx_hbm, table_hbm, out_hbm, idx_v, rows_a, rows_b, sem_a, sem_b):
    wid = lax.axis_index("s") * _NC + lax.axis_index("c")
    base = wid * _B_PER_W
    pltpu.sync_copy(idx_hbm.at[pl.ds(base, _B_PER_W)], idx_v)

    bufs = (rows_a, rows_b)
    sems = (sem_a, sem_b)
    copies = [None, None]

    copies[0] = pltpu.async_copy(
        table_hbm.at[idx_v.at[pl.ds(0, _CHUNK)]], bufs[0], sems[0])
    for c in range(_N_CHUNKS):
        cur = c % 2
        nxt = (c + 1) % 2
        if c + 1 < _N_CHUNKS:
            copies[nxt] = pltpu.async_copy(
                table_hbm.at[idx_v.at[pl.ds((c + 1) * _CHUNK, _CHUNK)]],
                bufs[nxt], sems[nxt])
        copies[cur].wait()
        pltpu.sync_copy(bufs[cur], out_hbm.at[pl.ds(base + c * _CHUNK, _CHUNK)])


_embed = functools.partial(
    pl.kernel,
    out_type=jax.ShapeDtypeStruct((B_TOTAL, OUTPUT_DIM), jnp.float32),
    mesh=plsc.VectorSubcoreMesh(core_axis_name="c", subcore_axis_name="s"),
    scratch_types=[
        pltpu.VMEM((_B_PER_W,), jnp.int32),
        pltpu.VMEM((_CHUNK, OUTPUT_DIM), jnp.float32),
        pltpu.VMEM((_CHUNK, OUTPUT_DIM), jnp.float32),
        pltpu.SemaphoreType.DMA,
        pltpu.SemaphoreType.DMA,
    ],
    compiler_params=pltpu.CompilerParams(use_tc_tiling_on_sc=False),
)(_embed_body)


@jax.jit
def kernel(indices, weight):
    flat_idx = indices.reshape(-1).astype(jnp.int32)
    lu = _embed(flat_idx, weight)
    return lu.reshape(indices.shape[0], indices.shape[1], OUTPUT_DIM), weight


# speedup vs baseline: 1.4675x; 1.0000x over previous
import functools

import jax
import jax.numpy as jnp
from jax import lax
from jax.experimental import pallas as pl
from jax.experimental.pallas import tpu as pltpu
from jax.experimental.pallas import tpu_sc as plsc

N_SYMBOLS = 1000000
OUTPUT_DIM = 32
B_TOTAL = 16384 * 20

_NC = 2
_NS = 16
_NW = _NC * _NS

_B_PER_W = B_TOTAL // _NW
_CHUNK = 512
_N_CHUNKS = _B_PER_W // _CHUNK
_NBUF = 4


def _embed_body(idx_hbm, table_hbm, out_hbm, idx_v, rows_bufs, gsems, wsems):
    wid = lax.axis_index("s") * _NC + lax.axis_index("c")
    base = wid * _B_PER_W
    pltpu.sync_copy(idx_hbm.at[pl.ds(base, _B_PER_W)], idx_v)

    g_copies = [None] * _NBUF
    w_copies = [None] * _NBUF

    for t in range(_N_CHUNKS + _NBUF - 1):
        if t < _N_CHUNKS:
            s = t % _NBUF
            if t >= _NBUF:
                w_copies[s].wait()
            g_copies[s] = pltpu.async_copy(
                table_hbm.at[idx_v.at[pl.ds(t * _CHUNK, _CHUNK)]],
                rows_bufs[s], gsems[s])
        d = t - (_NBUF - 1)
        if d >= 0:
            sd = d % _NBUF
            g_copies[sd].wait()
            w_copies[sd] = pltpu.async_copy(
                rows_bufs[sd], out_hbm.at[pl.ds(base + d * _CHUNK, _CHUNK)],
                wsems[sd])
    for d in range(max(0, _N_CHUNKS - _NBUF), _N_CHUNKS):
        w_copies[d % _NBUF].wait()


_embed = functools.partial(
    pl.kernel,
    out_type=jax.ShapeDtypeStruct((B_TOTAL, OUTPUT_DIM), jnp.float32),
    mesh=plsc.VectorSubcoreMesh(core_axis_name="c", subcore_axis_name="s"),
    scratch_types=[
        pltpu.VMEM((_B_PER_W,), jnp.int32),
        [pltpu.VMEM((_CHUNK, OUTPUT_DIM), jnp.float32) for _ in range(_NBUF)],
        [pltpu.SemaphoreType.DMA for _ in range(_NBUF)],
        [pltpu.SemaphoreType.DMA for _ in range(_NBUF)],
    ],
    compiler_params=pltpu.CompilerParams(use_tc_tiling_on_sc=False),
)(_embed_body)


@jax.jit
def kernel(indices, weight):
    flat_idx = indices.reshape(-1).astype(jnp.int32)
    lu = _embed(flat_idx, weight)
    return lu.reshape(indices.shape[0], indices.shape[1], OUTPUT_DIM), weight


# speedup vs baseline: 1.5502x; 1.0563x over previous
import functools

import jax
import jax.numpy as jnp
from jax import lax
from jax.experimental import pallas as pl
from jax.experimental.pallas import tpu as pltpu
from jax.experimental.pallas import tpu_sc as plsc

N_SYMBOLS = 1000000
OUTPUT_DIM = 32
B_TOTAL = 16384 * 20

_NC = 2
_NS = 16
_NW = _NC * _NS

_B_PER_W = B_TOTAL // _NW
_CHUNK = 512
_N_CHUNKS = _B_PER_W // _CHUNK
_NBUF = 4


def _embed_body(idx_hbm, table_hbm, out_hbm, idx_v, rows_bufs, gsems, wsems):
    wid = lax.axis_index("s") * _NC + lax.axis_index("c")
    base = wid * _B_PER_W
    pltpu.sync_copy(idx_hbm.at[pl.ds(base, _B_PER_W)], idx_v)

    g_copies = [None] * _NBUF
    w_copies = [None] * _NBUF

    for t in range(_N_CHUNKS + _NBUF - 1):
        if t < _N_CHUNKS:
            s = t % _NBUF
            if t >= _NBUF:
                w_copies[s].wait()
            g_copies[s] = pltpu.async_copy(
                table_hbm.at[idx_v.at[pl.ds(t * _CHUNK, _CHUNK)]],
                rows_bufs[s], gsems[s])
        d = t - (_NBUF - 1)
        if d >= 0:
            sd = d % _NBUF
            g_copies[sd].wait()
            w_copies[sd] = pltpu.async_copy(
                rows_bufs[sd], out_hbm.at[pl.ds(base + d * _CHUNK, _CHUNK)],
                wsems[sd])
    for d in range(max(0, _N_CHUNKS - _NBUF), _N_CHUNKS):
        w_copies[d % _NBUF].wait()


_embed = functools.partial(
    pl.kernel,
    out_type=jax.ShapeDtypeStruct((B_TOTAL, OUTPUT_DIM), jnp.float32),
    mesh=plsc.VectorSubcoreMesh(core_axis_name="c", subcore_axis_name="s"),
    scratch_types=[
        pltpu.VMEM((_B_PER_W,), jnp.int32),
        [pltpu.VMEM((_CHUNK, OUTPUT_DIM), jnp.float32) for _ in range(_NBUF)],
        [pltpu.SemaphoreType.DMA for _ in range(_NBUF)],
        [pltpu.SemaphoreType.DMA for _ in range(_NBUF)],
    ],
    compiler_params=pltpu.CompilerParams(use_tc_tiling_on_sc=False),
)(_embed_body)


@jax.jit
def kernel(indices, weight):
    idxt = jnp.transpose(indices, (1, 2, 0)).astype(jnp.int32)
    flat_idx = idxt.reshape(-1)
    lu = _embed(flat_idx, weight)
    lu = jnp.transpose(lu.reshape(indices.shape[1], indices.shape[0], OUTPUT_DIM),
                       (1, 0, 2))
    return lu, weight
